# baseline (speedup 1.0000x reference)
import functools

import jax
import jax.numpy as jnp
from jax import lax
from jax.experimental import pallas as pl
from jax.experimental.pallas import tpu as pltpu
from jax.experimental.pallas import tpu_sc as plsc

N = 10000
E = 320000
H = 128

NC = 2
NS = 16
NW = NC * NS
CH = 128
CPW = 80
EW = CPW * CH
E_PAD = NW * EW
NCHUNK = E_PAD // CH
N_PAD = 10240
DUMMY = N
NB = 2048
EBK = 2048
ROWS_PER_SUB = N_PAD // NS

_F32 = jnp.float32


def _elu(x):
    return jnp.where(x > 0, x, jnp.exp(jnp.minimum(x, 0.0)) - 1.0)


def _ln(y, g, b):
    m = jnp.mean(y, axis=-1, keepdims=True)
    v = jnp.mean((y - m) * (y - m), axis=-1, keepdims=True)
    return (y - m) * lax.rsqrt(v + 1e-5) * g + b


def _dot(a, b):
    return jnp.dot(a, b, preferred_element_type=_F32)


def _dot_bf(a, b):
    return jnp.dot(a.astype(jnp.bfloat16), b.astype(jnp.bfloat16),
                   preferred_element_type=_F32)


def _node_enc_body(x_ref, w1, b1, w2, b2, g, bt, wd, ws, pb, h_ref, p_ref, q_ref):
    y = _elu(_dot(x_ref[...], w1[...]) + b1[...])
    y = _dot(y, w2[...]) + b2[...]
    h = _ln(y, g[...], bt[...])
    h_ref[...] = h
    p_ref[...] = _dot(h, wd[...]) + pb[...]
    q_ref[...] = _dot(h, ws[...])


def _edge_enc_body(a_ref, w1, b1, w2, b2, g, bt, e_ref):
    y = _elu(_dot(a_ref[...], w1[...]) + b1[...])
    y = _dot(y, w2[...]) + b2[...]
    e_ref[...] = _ln(y, g[...], bt[...])


def _edge_upd_body(gp, gq, e_ref, we, w2, b2, g, bt, out_ref):
    z = _elu(gp[...] + gq[...] + _dot_bf(e_ref[...], we[...]))
    y = _dot_bf(z, w2[...]) + b2[...]
    out_ref[...] = e_ref[...] + _ln(y, g[...], bt[...])


def _node_upd_body(h_ref, s0, s1, c0, c1, wa, wb, bn1, wn2, bn2, g, bt,
                   wd, ws, pb, h_out, p_out, q_out):
    cnt = c0[:, :1] + c1[:, :1]
    inv = 1.0 / jnp.maximum(cnt, 1.0)
    agg = (s0[...] + s1[...]) * inv
    h = h_ref[...]
    u = _elu(_dot(h, wa[...]) + _dot(agg, wb[...]) + bn1[...])
    y = _dot(u, wn2[...]) + bn2[...]
    hn = h + _ln(y, g[...], bt[...])
    h_out[...] = hn
    p_out[...] = _dot(hn, wd[...]) + pb[...]
    q_out[...] = _dot(hn, ws[...])


def _full(shape):
    return pl.BlockSpec(shape, lambda i: (0,) * len(shape))


def _rows(blk, width):
    return pl.BlockSpec((blk, width), lambda i: (i, 0))


_TC_PARAMS = pltpu.CompilerParams(
    dimension_semantics=("arbitrary",),
)


def _node_enc(x, w1, b1, w2, b2, g, bt, wd, ws, pb):
    n_out = jax.ShapeDtypeStruct((N_PAD, H), _F32)
    return pl.pallas_call(
        _node_enc_body,
        grid=(N_PAD // NB,),
        in_specs=[_rows(NB, H)] + [_full((H, H))] + [_full((1, H))]
        + [_full((H, H))] + [_full((1, H))] * 3
        + [_full((H, H)), _full((H, H)), _full((1, H))],
        out_specs=[_rows(NB, H)] * 3,
        out_shape=[n_out] * 3,
        compiler_params=_TC_PARAMS,
    )(x, w1, b1, w2, b2, g, bt, wd, ws, pb)


def _edge_enc(a, w1, b1, w2, b2, g, bt):
    return pl.pallas_call(
        _edge_enc_body,
        grid=(E_PAD // EBK,),
        in_specs=[_rows(EBK, 16), _full((16, H)), _full((1, H)),
                  _full((H, H)), _full((1, H)), _full((1, H)), _full((1, H))],
        out_specs=_rows(EBK, H),
        out_shape=jax.ShapeDtypeStruct((E_PAD, H), _F32),
        compiler_params=_TC_PARAMS,
    )(a, w1, b1, w2, b2, g, bt)


def _edge_upd(gp, gq, e, we, w2, b2, g, bt):
    return pl.pallas_call(
        _edge_upd_body,
        grid=(E_PAD // EBK,),
        in_specs=[_rows(EBK, H)] * 3
        + [_full((H, H)), _full((H, H))] + [_full((1, H))] * 3,
        out_specs=_rows(EBK, H),
        out_shape=jax.ShapeDtypeStruct((E_PAD, H), _F32),
        compiler_params=_TC_PARAMS,
    )(gp, gq, e, we, w2, b2, g, bt)


def _node_upd(h, s0, s1, c0, c1, wa, wb, bn1, wn2, bn2, g, bt, wd, ws, pb):
    n_out = jax.ShapeDtypeStruct((N_PAD, H), _F32)
    return pl.pallas_call(
        _node_upd_body,
        grid=(N_PAD // NB,),
        in_specs=[_rows(NB, H)] * 3 + [_rows(NB, H)] * 2
        + [_full((H, H)), _full((H, H)), _full((1, H)),
           _full((H, H)), _full((1, H)), _full((1, H)), _full((1, H)),
           _full((H, H)), _full((H, H)), _full((1, H))],
        out_specs=[_rows(NB, H)] * 3,
        out_shape=[n_out] * 3,
        compiler_params=_TC_PARAMS,
    )(h, s0, s1, c0, c1, wa, wb, bn1, wn2, bn2, g, bt, wd, ws, pb)


@functools.cache
def _mesh():
    return plsc.VectorSubcoreMesh(
        core_axis_name="c", subcore_axis_name="s",
        num_cores=NC, num_subcores=NS)


def _wid():
    return lax.axis_index("s") * NC + lax.axis_index("c")


TG = 2
IG = 4
EW2 = E_PAD // NS
CP2 = EW2 // CH


def _sc_gather_body(p_hbm, q_hbm, dst1, src1, gp_hbm, gq_hbm,
                    shared, idx_r, buf, sid, sin, sog):
    c = lax.axis_index("c")
    s = lax.axis_index("s")

    @pl.when(c == 0)
    def _():
        pltpu.sync_copy(p_hbm.at[pl.ds(s * ROWS_PER_SUB, ROWS_PER_SUB)],
                        shared.at[pl.ds(s * ROWS_PER_SUB, ROWS_PER_SUB)])

    @pl.when(c == 1)
    def _():
        pltpu.sync_copy(q_hbm.at[pl.ds(s * ROWS_PER_SUB, ROWS_PER_SUB)],
                        shared.at[pl.ds(s * ROWS_PER_SUB, ROWS_PER_SUB)])

    plsc.subcore_barrier()

    def run(idx1, out_hbm):
        def idx_dma(j, b):
            return pltpu.make_async_copy(
                idx1.at[pl.ds(s * EW2 + j * CH, CH)], idx_r.at[b], sid.at[b])

        def in_t(j, b):
            return pltpu.make_async_copy(shared.at[idx_r.at[j % IG]],
                                         buf.at[b], sin.at[b])

        def out_g(j, b):
            base = s * EW2 + j * CH
            return pltpu.make_async_copy(buf.at[b],
                                         out_hbm.at[pl.ds(base, CH)],
                                         sog.at[b])

        for j0 in range(TG):
            idx_dma(j0, j0 % IG).start()
        for j0 in range(TG - 1):
            idx_dma(j0, j0 % IG).wait()
            in_t(j0, j0 % TG).start()

        def body(j, carry):
            b = j % TG

            @pl.when(j + TG < CP2)
            def _ipre():
                idx_dma(j + TG, (j + TG) % IG).start()

            @pl.when(j + TG - 1 < CP2)
            def _tpre():
                nb = (j + TG - 1) % TG

                @pl.when(j >= 1)
                def _free():
                    out_g(j - 1, nb).wait()

                idx_dma(j + TG - 1, (j + TG - 1) % IG).wait()
                in_t(j + TG - 1, nb).start()

            in_t(j, b).wait()
            out_g(j, b).start()
            return carry

        lax.fori_loop(0, CP2, body, 0)
        for j0 in range(CP2 - TG, CP2):
            out_g(j0, j0 % TG).wait()

    @pl.when(c == 0)
    def _():
        run(dst1, gp_hbm)

    @pl.when(c == 1)
    def _():
        run(src1, gq_hbm)


def _sc_gather(p, q, dst1, src1):
    out = jax.ShapeDtypeStruct((E_PAD, H), _F32)
    return pl.kernel(
        _sc_gather_body,
        out_type=(out, out),
        mesh=_mesh(),
        scratch_types=[
            pltpu.VMEM_SHARED((N_PAD, H), _F32),
            pltpu.VMEM((IG, CH), jnp.int32),
            pltpu.VMEM((TG, CH, H), _F32),
            pltpu.SemaphoreType.DMA((IG,)),
            pltpu.SemaphoreType.DMA((TG,)),
            pltpu.SemaphoreType.DMA((TG,)),
        ],
    )(p, q, dst1, src1)


SBUF = 2


def _sc_scatter_body(e_hbm, dsti, zeros_hbm, s_hbm, shared, idxd, vbuf, sin, ssc):
    c = lax.axis_index("c")
    s = lax.axis_index("s")
    w = s * NC + c
    r0 = s * ROWS_PER_SUB
    pltpu.sync_copy(zeros_hbm.at[pl.ds(r0, ROWS_PER_SUB)],
                    shared.at[pl.ds(r0, ROWS_PER_SUB)])
    pltpu.sync_copy(dsti.at[pl.ds(w * CPW, CPW)], idxd)
    plsc.subcore_barrier()

    def in_e(j, b):
        base = w * EW + j * CH
        return pltpu.make_async_copy(e_hbm.at[pl.ds(base, CH)], vbuf.at[b],
                                     sin.at[b])

    def wait_sc(j, b):
        pltpu.make_async_copy(vbuf.at[b], shared.at[idxd.at[j]],
                              ssc.at[b]).wait()

    for j0 in range(SBUF - 1):
        in_e(j0, j0).start()

    def body(j, carry):
        b = j % SBUF
        nb = (j + SBUF - 1) % SBUF

        @pl.when(j + SBUF - 1 < CPW)
        def _prefetch():
            @pl.when(j >= 1)
            def _free():
                wait_sc(j - 1, nb)
            in_e(j + SBUF - 1, nb).start()

        in_e(j, b).wait()
        pltpu.async_copy(vbuf.at[b], shared.at[idxd.at[j]], ssc.at[b],
                         add=True)
        return carry

    lax.fori_loop(0, CPW, body, 0)
    for j0 in range(CPW - SBUF, CPW):
        if j0 >= 0:
            wait_sc(j0, j0 % SBUF)
    plsc.subcore_barrier()
    pltpu.sync_copy(shared.at[pl.ds(r0, ROWS_PER_SUB)],
                    s_hbm.at[c].at[pl.ds(r0, ROWS_PER_SUB)])


def _sc_scatter(e, dsti, zeros_hbm):
    return pl.kernel(
        _sc_scatter_body,
        out_type=jax.ShapeDtypeStruct((NC, N_PAD, H), _F32),
        mesh=_mesh(),
        scratch_types=[
            pltpu.VMEM_SHARED((N_PAD, H), _F32),
            pltpu.VMEM((CPW, CH), jnp.int32),
            pltpu.VMEM((SBUF, CH, H), _F32),
            pltpu.SemaphoreType.DMA((SBUF,)),
            pltpu.SemaphoreType.DMA((SBUF,)),
        ],
    )(e, dsti, zeros_hbm)


def _sc_count_body(dsti, zeros_hbm, ones_hbm, c_hbm, shared, idxd, vbuf, sem):
    c = lax.axis_index("c")
    s = lax.axis_index("s")
    w = s * NC + c
    r0 = s * ROWS_PER_SUB
    pltpu.sync_copy(zeros_hbm.at[pl.ds(r0, ROWS_PER_SUB)],
                    shared.at[pl.ds(r0, ROWS_PER_SUB)])
    pltpu.sync_copy(dsti.at[pl.ds(w * CPW, CPW)], idxd)
    pltpu.sync_copy(ones_hbm, vbuf)
    plsc.subcore_barrier()

    def body(j, carry):
        pltpu.sync_copy(vbuf, shared.at[idxd.at[j]], add=True)
        return carry

    lax.fori_loop(0, CPW, body, 0)
    plsc.subcore_barrier()
    pltpu.sync_copy(shared.at[pl.ds(r0, ROWS_PER_SUB)],
                    c_hbm.at[c].at[pl.ds(r0, ROWS_PER_SUB)])


def _sc_count(dsti, zeros_hbm, ones_hbm):
    return pl.kernel(
        _sc_count_body,
        out_type=jax.ShapeDtypeStruct((NC, N_PAD, H), _F32),
        mesh=_mesh(),
        scratch_types=[
            pltpu.VMEM_SHARED((N_PAD, H), _F32),
            pltpu.VMEM((CPW, CH), jnp.int32),
            pltpu.VMEM((CH, H), _F32),
            pltpu.SemaphoreType.DMA,
        ],
    )(dsti, zeros_hbm, ones_hbm)


def kernel(x, edge_attr, params, edge_index):
    x_pad = jnp.zeros((N_PAD, H), _F32).at[:N].set(x)
    ea_pad = jnp.zeros((E_PAD, 16), _F32).at[:E].set(edge_attr)
    dst = jnp.full((E_PAD,), DUMMY, jnp.int32).at[:E].set(edge_index[1])
    src = jnp.full((E_PAD,), DUMMY, jnp.int32).at[:E].set(edge_index[0])
    dsti = dst.reshape(NCHUNK, CH)
    srci = src.reshape(NCHUNK, CH)
    zeros_hbm = jnp.zeros((N_PAD, H), _F32)
    ones_rows = jnp.ones((CH, H), _F32)

    blocks = list(params["down"]) + list(params["up"])
    b0 = blocks[0]

    def split(blk):
        w1 = blk["e_ws"][0]
        wn1 = blk["n_ws"][0]
        return dict(
            wd=w1[:H], ws=w1[H:2 * H], we=w1[2 * H:],
            eb1=blk["e_bs"][0][None, :],
            ew2=blk["e_ws"][1], eb2=blk["e_bs"][1][None, :],
            eg=blk["e_g"][None, :], ebt=blk["e_b"][None, :],
            wa=wn1[:H], wb=wn1[H:],
            nb1=blk["n_bs"][0][None, :],
            nw2=blk["n_ws"][1], nb2=blk["n_bs"][1][None, :],
            ng=blk["n_g"][None, :], nbt=blk["n_b"][None, :],
        )

    sp = [split(b) for b in blocks]

    h, p, q = _node_enc(
        x_pad,
        params["node_enc_ws"][0], params["node_enc_bs"][0][None, :],
        params["node_enc_ws"][1], params["node_enc_bs"][1][None, :],
        params["node_enc_g"][None, :], params["node_enc_b"][None, :],
        sp[0]["wd"], sp[0]["ws"], sp[0]["eb1"])

    e = _edge_enc(
        ea_pad,
        params["edge_enc_ws"][0], params["edge_enc_bs"][0][None, :],
        params["edge_enc_ws"][1], params["edge_enc_bs"][1][None, :],
        params["edge_enc_g"][None, :], params["edge_enc_b"][None, :])

    cparts = _sc_count(dsti, zeros_hbm, ones_rows)
    c0, c1 = cparts[0], cparts[1]

    nblk = len(blocks)
    for i in range(nblk):
        s = sp[i]
        gp, gq = _sc_gather(p, q, dst, src)
        e = _edge_upd(gp, gq, e, s["we"], s["ew2"], s["eb2"], s["eg"], s["ebt"])
        parts = _sc_scatter(e, dsti, zeros_hbm)
        nxt = sp[i + 1] if i + 1 < nblk else sp[0]
        h, p, q = _node_upd(
            h, parts[0], parts[1], c0, c1,
            s["wa"], s["wb"], s["nb1"], s["nw2"], s["nb2"], s["ng"], s["nbt"],
            nxt["wd"], nxt["ws"], nxt["eb1"])

    return h[:N]


# speedup vs baseline: 1.0154x; 1.0154x over previous
import functools

import jax
import jax.numpy as jnp
from jax import lax
from jax.experimental import pallas as pl
from jax.experimental.pallas import tpu as pltpu
from jax.experimental.pallas import tpu_sc as plsc

N = 10000
E = 320000
H = 128

NC = 2
NS = 16
NW = NC * NS
CH = 128
CPW = 79
EW = CPW * CH
E_PAD = NW * EW
NCHUNK = E_PAD // CH
N_PAD = 10240
DUMMY = N
NB = 2048
EBK = 2048
ROWS_PER_SUB = N_PAD // NS

_F32 = jnp.float32


def _elu(x):
    return jnp.where(x > 0, x, jnp.exp(jnp.minimum(x, 0.0)) - 1.0)


def _ln(y, g, b):
    m = jnp.mean(y, axis=-1, keepdims=True)
    v = jnp.mean((y - m) * (y - m), axis=-1, keepdims=True)
    return (y - m) * lax.rsqrt(v + 1e-5) * g + b


def _dot(a, b):
    return jnp.dot(a, b, preferred_element_type=_F32)


def _dot_bf(a, b):
    return jnp.dot(a.astype(jnp.bfloat16), b.astype(jnp.bfloat16),
                   preferred_element_type=_F32)


def _node_enc_body(x_ref, w1, b1, w2, b2, g, bt, wd, ws, pb, h_ref, p_ref, q_ref):
    y = _elu(_dot(x_ref[...], w1[...]) + b1[...])
    y = _dot(y, w2[...]) + b2[...]
    h = _ln(y, g[...], bt[...])
    h_ref[...] = h
    p_ref[...] = _dot(h, wd[...]) + pb[...]
    q_ref[...] = _dot(h, ws[...])


def _edge_enc_body(a_ref, w1, b1, w2, b2, g, bt, e_ref):
    y = _elu(_dot(a_ref[...], w1[...]) + b1[...])
    y = _dot(y, w2[...]) + b2[...]
    e_ref[...] = _ln(y, g[...], bt[...])


def _edge_upd_body(gp, gq, e_ref, we, w2, b2, g, bt, out_ref):
    z = _elu(gp[...] + gq[...] + _dot(e_ref[...], we[...]))
    y = _dot(z, w2[...]) + b2[...]
    out_ref[...] = e_ref[...] + _ln(y, g[...], bt[...])


def _node_upd_body(h_ref, s0, s1, c0, c1, wa, wb, bn1, wn2, bn2, g, bt,
                   wd, ws, pb, h_out, p_out, q_out):
    cnt = c0[:, :1] + c1[:, :1]
    inv = 1.0 / jnp.maximum(cnt, 1.0)
    agg = (s0[...] + s1[...]) * inv
    h = h_ref[...]
    u = _elu(_dot(h, wa[...]) + _dot(agg, wb[...]) + bn1[...])
    y = _dot(u, wn2[...]) + bn2[...]
    hn = h + _ln(y, g[...], bt[...])
    h_out[...] = hn
    p_out[...] = _dot(hn, wd[...]) + pb[...]
    q_out[...] = _dot(hn, ws[...])


def _full(shape):
    return pl.BlockSpec(shape, lambda i: (0,) * len(shape))


def _rows(blk, width):
    return pl.BlockSpec((blk, width), lambda i: (i, 0))


_TC_PARAMS = pltpu.CompilerParams(
    dimension_semantics=("arbitrary",),
)


def _node_enc(x, w1, b1, w2, b2, g, bt, wd, ws, pb):
    n_out = jax.ShapeDtypeStruct((N_PAD, H), _F32)
    return pl.pallas_call(
        _node_enc_body,
        grid=(N_PAD // NB,),
        in_specs=[_rows(NB, H)] + [_full((H, H))] + [_full((1, H))]
        + [_full((H, H))] + [_full((1, H))] * 3
        + [_full((H, H)), _full((H, H)), _full((1, H))],
        out_specs=[_rows(NB, H)] * 3,
        out_shape=[n_out] * 3,
        compiler_params=_TC_PARAMS,
    )(x, w1, b1, w2, b2, g, bt, wd, ws, pb)


def _edge_enc(a, w1, b1, w2, b2, g, bt):
    return pl.pallas_call(
        _edge_enc_body,
        grid=(E_PAD // EBK,),
        in_specs=[_rows(EBK, 16), _full((16, H)), _full((1, H)),
                  _full((H, H)), _full((1, H)), _full((1, H)), _full((1, H))],
        out_specs=_rows(EBK, H),
        out_shape=jax.ShapeDtypeStruct((E_PAD, H), _F32),
        compiler_params=_TC_PARAMS,
    )(a, w1, b1, w2, b2, g, bt)


def _edge_upd(gp, gq, e, we, w2, b2, g, bt):
    return pl.pallas_call(
        _edge_upd_body,
        grid=(E_PAD // EBK,),
        in_specs=[_rows(EBK, H)] * 3
        + [_full((H, H)), _full((H, H))] + [_full((1, H))] * 3,
        out_specs=_rows(EBK, H),
        out_shape=jax.ShapeDtypeStruct((E_PAD, H), _F32),
        compiler_params=_TC_PARAMS,
    )(gp, gq, e, we, w2, b2, g, bt)


def _node_upd(h, s0, s1, c0, c1, wa, wb, bn1, wn2, bn2, g, bt, wd, ws, pb):
    n_out = jax.ShapeDtypeStruct((N_PAD, H), _F32)
    return pl.pallas_call(
        _node_upd_body,
        grid=(N_PAD // NB,),
        in_specs=[_rows(NB, H)] * 3 + [_rows(NB, H)] * 2
        + [_full((H, H)), _full((H, H)), _full((1, H)),
           _full((H, H)), _full((1, H)), _full((1, H)), _full((1, H)),
           _full((H, H)), _full((H, H)), _full((1, H))],
        out_specs=[_rows(NB, H)] * 3,
        out_shape=[n_out] * 3,
        compiler_params=_TC_PARAMS,
    )(h, s0, s1, c0, c1, wa, wb, bn1, wn2, bn2, g, bt, wd, ws, pb)


@functools.cache
def _mesh():
    return plsc.VectorSubcoreMesh(
        core_axis_name="c", subcore_axis_name="s",
        num_cores=NC, num_subcores=NS)


def _wid():
    return lax.axis_index("s") * NC + lax.axis_index("c")


TG = 2
IG = 4
EW2 = E_PAD // NS
CP2 = EW2 // CH


def _sc_gather_body(p_hbm, q_hbm, dst1, src1, gp_hbm, gq_hbm,
                    shared, idx_r, buf, sid, sin, sog):
    c = lax.axis_index("c")
    s = lax.axis_index("s")

    @pl.when(c == 0)
    def _():
        pltpu.sync_copy(p_hbm.at[pl.ds(s * ROWS_PER_SUB, ROWS_PER_SUB)],
                        shared.at[pl.ds(s * ROWS_PER_SUB, ROWS_PER_SUB)])

    @pl.when(c == 1)
    def _():
        pltpu.sync_copy(q_hbm.at[pl.ds(s * ROWS_PER_SUB, ROWS_PER_SUB)],
                        shared.at[pl.ds(s * ROWS_PER_SUB, ROWS_PER_SUB)])

    plsc.subcore_barrier()

    def run(idx1, out_hbm):
        def idx_dma(j, b):
            return pltpu.make_async_copy(
                idx1.at[pl.ds(s * EW2 + j * CH, CH)], idx_r.at[b], sid.at[b])

        def in_t(j, b):
            return pltpu.make_async_copy(shared.at[idx_r.at[j % IG]],
                                         buf.at[b], sin.at[b])

        def out_g(j, b):
            base = s * EW2 + j * CH
            return pltpu.make_async_copy(buf.at[b],
                                         out_hbm.at[pl.ds(base, CH)],
                                         sog.at[b])

        for j0 in range(TG):
            idx_dma(j0, j0 % IG).start()
        for j0 in range(TG - 1):
            idx_dma(j0, j0 % IG).wait()
            in_t(j0, j0 % TG).start()

        def body(j, carry):
            b = j % TG

            @pl.when(j + TG < CP2)
            def _ipre():
                idx_dma(j + TG, (j + TG) % IG).start()

            @pl.when(j + TG - 1 < CP2)
            def _tpre():
                nb = (j + TG - 1) % TG

                @pl.when(j >= 1)
                def _free():
                    out_g(j - 1, nb).wait()

                idx_dma(j + TG - 1, (j + TG - 1) % IG).wait()
                in_t(j + TG - 1, nb).start()

            in_t(j, b).wait()
            out_g(j, b).start()
            return carry

        lax.fori_loop(0, CP2, body, 0)
        for j0 in range(CP2 - TG, CP2):
            out_g(j0, j0 % TG).wait()

    @pl.when(c == 0)
    def _():
        run(dst1, gp_hbm)

    @pl.when(c == 1)
    def _():
        run(src1, gq_hbm)


def _sc_gather(p, q, dst1, src1):
    out = jax.ShapeDtypeStruct((E_PAD, H), _F32)
    return pl.kernel(
        _sc_gather_body,
        out_type=(out, out),
        mesh=_mesh(),
        scratch_types=[
            pltpu.VMEM_SHARED((N_PAD, H), _F32),
            pltpu.VMEM((IG, CH), jnp.int32),
            pltpu.VMEM((TG, CH, H), _F32),
            pltpu.SemaphoreType.DMA((IG,)),
            pltpu.SemaphoreType.DMA((TG,)),
            pltpu.SemaphoreType.DMA((TG,)),
        ],
    )(p, q, dst1, src1)


SBUF = 2


def _sc_scatter_body(e_hbm, dsti, zeros_hbm, s_hbm, shared, idxd, vbuf, sin, ssc):
    c = lax.axis_index("c")
    s = lax.axis_index("s")
    w = s * NC + c
    r0 = s * ROWS_PER_SUB
    pltpu.sync_copy(zeros_hbm.at[pl.ds(r0, ROWS_PER_SUB)],
                    shared.at[pl.ds(r0, ROWS_PER_SUB)])
    pltpu.sync_copy(dsti.at[w], idxd)
    plsc.subcore_barrier()

    def in_e(j, b):
        base = w * EW + j * CH
        return pltpu.make_async_copy(e_hbm.at[pl.ds(base, CH)], vbuf.at[b],
                                     sin.at[b])

    def wait_sc(j, b):
        pltpu.make_async_copy(vbuf.at[b], shared.at[idxd.at[j]],
                              ssc.at[b]).wait()

    for j0 in range(SBUF - 1):
        in_e(j0, j0).start()

    def body(j, carry):
        b = j % SBUF
        nb = (j + SBUF - 1) % SBUF

        @pl.when(j + SBUF - 1 < CPW)
        def _prefetch():
            @pl.when(j >= 1)
            def _free():
                wait_sc(j - 1, nb)
            in_e(j + SBUF - 1, nb).start()

        in_e(j, b).wait()
        pltpu.async_copy(vbuf.at[b], shared.at[idxd.at[j]], ssc.at[b],
                         add=True)
        return carry

    lax.fori_loop(0, CPW, body, 0)
    for j0 in range(CPW - SBUF, CPW):
        if j0 >= 0:
            wait_sc(j0, j0 % SBUF)
    plsc.subcore_barrier()
    pltpu.sync_copy(shared.at[pl.ds(r0, ROWS_PER_SUB)],
                    s_hbm.at[c].at[pl.ds(r0, ROWS_PER_SUB)])


def _sc_scatter(e, dsti, zeros_hbm):
    return pl.kernel(
        _sc_scatter_body,
        out_type=jax.ShapeDtypeStruct((NC, N_PAD, H), _F32),
        mesh=_mesh(),
        scratch_types=[
            pltpu.VMEM_SHARED((N_PAD, H), _F32),
            pltpu.VMEM((CPW, CH), jnp.int32),
            pltpu.VMEM((SBUF, CH, H), _F32),
            pltpu.SemaphoreType.DMA((SBUF,)),
            pltpu.SemaphoreType.DMA((SBUF,)),
        ],
    )(e, dsti, zeros_hbm)


def _sc_count_body(dsti, zeros_hbm, ones_hbm, c_hbm, shared, idxd, vbuf, sem):
    c = lax.axis_index("c")
    s = lax.axis_index("s")
    w = s * NC + c
    r0 = s * ROWS_PER_SUB
    pltpu.sync_copy(zeros_hbm.at[pl.ds(r0, ROWS_PER_SUB)],
                    shared.at[pl.ds(r0, ROWS_PER_SUB)])
    pltpu.sync_copy(dsti.at[w], idxd)
    pltpu.sync_copy(ones_hbm, vbuf)
    plsc.subcore_barrier()

    def body(j, carry):
        pltpu.sync_copy(vbuf, shared.at[idxd.at[j]], add=True)
        return carry

    lax.fori_loop(0, CPW, body, 0)
    plsc.subcore_barrier()
    pltpu.sync_copy(shared.at[pl.ds(r0, ROWS_PER_SUB)],
                    c_hbm.at[c].at[pl.ds(r0, ROWS_PER_SUB)])


def _sc_count(dsti, zeros_hbm, ones_hbm):
    return pl.kernel(
        _sc_count_body,
        out_type=jax.ShapeDtypeStruct((NC, N_PAD, H), _F32),
        mesh=_mesh(),
        scratch_types=[
            pltpu.VMEM_SHARED((N_PAD, H), _F32),
            pltpu.VMEM((CPW, CH), jnp.int32),
            pltpu.VMEM((CH, H), _F32),
            pltpu.SemaphoreType.DMA,
        ],
    )(dsti, zeros_hbm, ones_hbm)


def kernel(x, edge_attr, params, edge_index):
    x_pad = jnp.zeros((N_PAD, H), _F32).at[:N].set(x)
    ea_pad = jnp.zeros((E_PAD, 16), _F32).at[:E].set(edge_attr)
    dst = jnp.full((E_PAD,), DUMMY, jnp.int32).at[:E].set(edge_index[1])
    src = jnp.full((E_PAD,), DUMMY, jnp.int32).at[:E].set(edge_index[0])
    dsti = dst.reshape(NW, CPW, CH)
    zeros_hbm = jnp.zeros((N_PAD, H), _F32)
    ones_rows = jnp.ones((CH, H), _F32)

    blocks = list(params["down"]) + list(params["up"])
    b0 = blocks[0]

    def split(blk):
        w1 = blk["e_ws"][0]
        wn1 = blk["n_ws"][0]
        return dict(
            wd=w1[:H], ws=w1[H:2 * H], we=w1[2 * H:],
            eb1=blk["e_bs"][0][None, :],
            ew2=blk["e_ws"][1], eb2=blk["e_bs"][1][None, :],
            eg=blk["e_g"][None, :], ebt=blk["e_b"][None, :],
            wa=wn1[:H], wb=wn1[H:],
            nb1=blk["n_bs"][0][None, :],
            nw2=blk["n_ws"][1], nb2=blk["n_bs"][1][None, :],
            ng=blk["n_g"][None, :], nbt=blk["n_b"][None, :],
        )

    sp = [split(b) for b in blocks]

    h, p, q = _node_enc(
        x_pad,
        params["node_enc_ws"][0], params["node_enc_bs"][0][None, :],
        params["node_enc_ws"][1], params["node_enc_bs"][1][None, :],
        params["node_enc_g"][None, :], params["node_enc_b"][None, :],
        sp[0]["wd"], sp[0]["ws"], sp[0]["eb1"])

    e = _edge_enc(
        ea_pad,
        params["edge_enc_ws"][0], params["edge_enc_bs"][0][None, :],
        params["edge_enc_ws"][1], params["edge_enc_bs"][1][None, :],
        params["edge_enc_g"][None, :], params["edge_enc_b"][None, :])

    cparts = _sc_count(dsti, zeros_hbm, ones_rows)
    c0, c1 = cparts[0], cparts[1]

    nblk = len(blocks)
    for i in range(nblk):
        s = sp[i]
        gp, gq = _sc_gather(p, q, dst, src)
        e = _edge_upd(gp, gq, e, s["we"], s["ew2"], s["eb2"], s["eg"], s["ebt"])
        parts = _sc_scatter(e, dsti, zeros_hbm)
        nxt = sp[i + 1] if i + 1 < nblk else sp[0]
        h, p, q = _node_upd(
            h, parts[0], parts[1], c0, c1,
            s["wa"], s["wb"], s["nb1"], s["nw2"], s["nb2"], s["ng"], s["nbt"],
            nxt["wd"], nxt["ws"], nxt["eb1"])

    return h[:N]
